# baseline (speedup 1.0000x reference)
import functools

import jax
import jax.numpy as jnp
from jax import lax
from jax.experimental import pallas as pl
from jax.experimental.pallas import tpu as pltpu
from jax.experimental.pallas import tpu_sc as plsc

E = 8
TOP_K = 2
N_GROUP = 4
TOPK_GROUP = 2
SCALE = 2.5

R = 256
RS = 256
NC = 2
NS = 16
NW = NC * NS


def _route(x_flat, gate_weight, bias):
    router_logits = x_flat.astype(jnp.float32) @ gate_weight.T
    scores = jax.nn.sigmoid(router_logits)
    scores_for_choice = scores + bias[None, :]
    t = scores_for_choice.shape[0]
    grp = scores_for_choice.reshape(t, N_GROUP, E // N_GROUP)
    top2_vals, _ = jax.lax.top_k(grp, 2)
    group_scores = top2_vals.sum(axis=-1)
    _, group_idx = jax.lax.top_k(group_scores, TOPK_GROUP)
    ar = jnp.arange(N_GROUP, dtype=group_idx.dtype)[None, :]
    group_mask = jnp.where(
        (group_idx[:, 0:1] == ar) | (group_idx[:, 1:2] == ar), 1.0, 0.0
    ).astype(group_scores.dtype)
    score_mask = jnp.repeat(group_mask, E // N_GROUP, axis=1)
    masked_scores = jnp.where(score_mask > 0, scores_for_choice, 0.0)
    _, topk_idx = jax.lax.top_k(masked_scores, TOP_K)
    topk_w = jnp.take_along_axis(scores, topk_idx, axis=1)
    topk_w = topk_w / (topk_w.sum(axis=-1, keepdims=True) + 1e-20)
    topk_w = topk_w * SCALE
    return topk_idx, topk_w


def _dispatch_sc(x2, i0, i1, p_max):
    t = x2.shape[0] // 16
    per_w = t // NW
    chtok = 32
    n_ch = per_w // chtok
    nj = chtok * 16 // 128
    mesh = plsc.VectorSubcoreMesh(core_axis_name="c", subcore_axis_name="s")

    @functools.partial(
        pl.kernel, mesh=mesh,
        out_type=jax.ShapeDtypeStruct((p_max * 16, 128), jnp.float32),
        scratch_types=[
            pltpu.VMEM((per_w * 16 // 128, 128), jnp.int32),
            pltpu.VMEM((per_w * 16 // 128, 128), jnp.int32),
            pltpu.VMEM((chtok * 16, 128), jnp.float32),
            pltpu.SemaphoreType.DMA,
            pltpu.SemaphoreType.DMA,
        ],
    )
    def run(x_hbm, i0_hbm, i1_hbm, xg_hbm, idx0, idx1, rows, sem0, sem1):
        wid = lax.axis_index("s") * NC + lax.axis_index("c")
        irows = per_w * 16 // 128
        pltpu.sync_copy(i0_hbm.at[pl.ds(wid * irows, irows)], idx0)
        pltpu.sync_copy(i1_hbm.at[pl.ds(wid * irows, irows)], idx1)
        for c in range(n_ch):
            base = (wid * per_w + c * chtok) * 16
            pltpu.sync_copy(x_hbm.at[pl.ds(base, chtok * 16)], rows)
            cps = []
            for j in range(nj):
                seg = rows.at[pl.ds(j * 128, 128)]
                jj = c * nj + j
                cps.append(pltpu.async_copy(seg, xg_hbm.at[idx0.at[jj]], sem0))
                cps.append(pltpu.async_copy(seg, xg_hbm.at[idx1.at[jj]], sem1))
            for cp in cps:
                cp.wait()

    return run(x2, i0, i1)


def _combine_sc(yg2, i0, i1):
    t = i0.shape[0] * 128 // 16
    per_w = t // NW
    chtok = 16
    n_ch = per_w // chtok
    nj = chtok * 16 // 128
    mesh = plsc.VectorSubcoreMesh(core_axis_name="c", subcore_axis_name="s")
    oshape = jax.ShapeDtypeStruct((t * 16, 128), jnp.float32)

    @functools.partial(
        pl.kernel, mesh=mesh,
        out_type=(oshape, oshape),
        scratch_types=[
            pltpu.VMEM((per_w * 16 // 128, 128), jnp.int32),
            pltpu.VMEM((per_w * 16 // 128, 128), jnp.int32),
            pltpu.VMEM((chtok * 16, 128), jnp.float32),
            pltpu.VMEM((chtok * 16, 128), jnp.float32),
            pltpu.SemaphoreType.DMA,
            pltpu.SemaphoreType.DMA,
        ],
    )
    def run(yg_hbm, i0_hbm, i1_hbm, y0_hbm, y1_hbm, idx0, idx1, rows0, rows1,
            sem0, sem1):
        wid = lax.axis_index("s") * NC + lax.axis_index("c")
        irows = per_w * 16 // 128
        pltpu.sync_copy(i0_hbm.at[pl.ds(wid * irows, irows)], idx0)
        pltpu.sync_copy(i1_hbm.at[pl.ds(wid * irows, irows)], idx1)
        for c in range(n_ch):
            base = (wid * per_w + c * chtok) * 16
            cps = []
            for j in range(nj):
                sl = pl.ds(j * 128, 128)
                jj = c * nj + j
                cps.append(
                    pltpu.async_copy(yg_hbm.at[idx0.at[jj]], rows0.at[sl], sem0))
                cps.append(
                    pltpu.async_copy(yg_hbm.at[idx1.at[jj]], rows1.at[sl], sem1))
            for cp in cps:
                cp.wait()
            pltpu.sync_copy(rows0, y0_hbm.at[pl.ds(base, chtok * 16)])
            pltpu.sync_copy(rows1, y1_hbm.at[pl.ds(base, chtok * 16)])

    return run(yg2, i0, i1)


def _reformat_body(x_ref, out_ref):
    out_ref[...] = x_ref[...].reshape(out_ref.shape)


def _gmm_body(te_ref, xg_ref, w1_ref, w3_ref, w2_ref, out_ref):
    rdim = out_ref.shape[0] // 16
    hdim = w1_ref.shape[2]
    xr = xg_ref[...].reshape(rdim, hdim).astype(jnp.bfloat16)
    w1 = w1_ref[0].astype(jnp.bfloat16)
    w3 = w3_ref[0].astype(jnp.bfloat16)
    w2 = w2_ref[0].astype(jnp.bfloat16)
    nt = (((1,), (1,)), ((), ()))
    a = jax.lax.dot_general(xr, w1, nt, preferred_element_type=jnp.float32)
    b = jax.lax.dot_general(xr, w3, nt, preferred_element_type=jnp.float32)
    h = ((a * jax.nn.sigmoid(a)) * b).astype(jnp.bfloat16)
    y = jax.lax.dot_general(h, w2, nt, preferred_element_type=jnp.float32)
    out_ref[...] = y.reshape(out_ref.shape)


def _shared_body(x_ref, ws1_ref, ws3_ref, ws2_ref, out_ref):
    rs = out_ref.shape[0] // 16
    hdim = ws1_ref.shape[1]
    xr = x_ref[...].reshape(rs, hdim).astype(jnp.bfloat16)
    w1 = ws1_ref[...].astype(jnp.bfloat16)
    w3 = ws3_ref[...].astype(jnp.bfloat16)
    w2 = ws2_ref[...].astype(jnp.bfloat16)
    nt = (((1,), (1,)), ((), ()))
    a = jax.lax.dot_general(xr, w1, nt, preferred_element_type=jnp.float32)
    b = jax.lax.dot_general(xr, w3, nt, preferred_element_type=jnp.float32)
    h = ((a * jax.nn.sigmoid(a)) * b).astype(jnp.bfloat16)
    y = jax.lax.dot_general(h, w2, nt, preferred_element_type=jnp.float32)
    out_ref[...] = y.reshape(out_ref.shape)


def _add_body(sha_ref, shb_ref, y0_ref, y1_ref, w0_ref, w1c_ref, out_ref):
    rs, hdim = out_ref.shape
    half = pl.num_programs(0) // 2
    sh = jnp.where(pl.program_id(0) < half,
                   sha_ref[...], shb_ref[...]).reshape(rs, hdim)
    y = sh + w0_ref[...] * y0_ref[...].reshape(rs, hdim)
    y = y + w1c_ref[...] * y1_ref[...].reshape(rs, hdim)
    out_ref[...] = y


def kernel(hidden_states, gate_weight, e_score_correction_bias,
           w1, w2, w3, ws1, ws2, ws3):
    b, s, hdim = hidden_states.shape
    t = b * s
    d_ff = w1.shape[1]
    d_sh = ws1.shape[0]
    x = hidden_states.reshape(t, hdim)

    topk_idx, topk_w = _route(x, gate_weight, e_score_correction_bias)

    p_max = t * TOP_K + E * R
    n_tiles = p_max // R
    onehot = jax.nn.one_hot(topk_idx, E, dtype=jnp.int32).sum(axis=1)
    rank = jnp.cumsum(onehot, axis=0) - onehot
    counts = jnp.sum(onehot, axis=0)
    padded = ((counts + R - 1) // R) * R
    pad_off = jnp.concatenate(
        [jnp.zeros((1,), jnp.int32), jnp.cumsum(padded)[:-1].astype(jnp.int32)])
    slots = jnp.take_along_axis(pad_off[None, :] + rank, topk_idx, axis=1)

    def strip_idx(sv):
        return (sv[:, None] * 16
                + jnp.arange(16, dtype=jnp.int32)[None, :]
                ).astype(jnp.int32).reshape(t * 16 // 128, 128)

    i0 = strip_idx(slots[:, 0])
    i1 = strip_idx(slots[:, 1])
    tile_expert = (jnp.sum(
        pad_off[None, :] <= (jnp.arange(n_tiles, dtype=jnp.int32) * R)[:, None],
        axis=1) - 1).astype(jnp.int32)

    x2 = pl.pallas_call(
        _reformat_body,
        grid=(t // 512,),
        in_specs=[pl.BlockSpec((512, hdim), lambda i: (i, 0))],
        out_specs=pl.BlockSpec((512 * 16, 128), lambda i: (i, 0)),
        out_shape=jax.ShapeDtypeStruct((t * 16, 128), jnp.float32),
    )(x)

    nh = t // 2 // RS

    def _shared_half(off):
        return pl.pallas_call(
            _shared_body,
            grid=(nh,),
            in_specs=[
                pl.BlockSpec((RS * 16, 128), lambda i: (i + off, 0)),
                pl.BlockSpec((d_sh, hdim), lambda i: (0, 0)),
                pl.BlockSpec((d_sh, hdim), lambda i: (0, 0)),
                pl.BlockSpec((hdim, d_sh), lambda i: (0, 0)),
            ],
            out_specs=pl.BlockSpec((RS * 16, 128), lambda i: (i, 0)),
            out_shape=jax.ShapeDtypeStruct((t * 16 // 2, 128), jnp.float32),
        )(x2, ws1, ws3, ws2)

    xg2 = _dispatch_sc(x2, i0, i1, p_max)
    shared_a = _shared_half(0)

    yg = pl.pallas_call(
        _gmm_body,
        grid_spec=pltpu.PrefetchScalarGridSpec(
            num_scalar_prefetch=1,
            grid=(n_tiles,),
            in_specs=[
                pl.BlockSpec((R * 16, 128), lambda i, te: (i, 0)),
                pl.BlockSpec((1, d_ff, hdim), lambda i, te: (te[i], 0, 0)),
                pl.BlockSpec((1, d_ff, hdim), lambda i, te: (te[i], 0, 0)),
                pl.BlockSpec((1, hdim, d_ff), lambda i, te: (te[i], 0, 0)),
            ],
            out_specs=pl.BlockSpec((R * 16, 128), lambda i, te: (i, 0)),
        ),
        out_shape=jax.ShapeDtypeStruct((p_max * 16, 128), jnp.float32),
    )(tile_expert, xg2, w1, w3, w2)

    y0, y1 = _combine_sc(yg, i0, i1)
    shared_b = _shared_half(nh)

    out = pl.pallas_call(
        _add_body,
        grid=(t // RS,),
        in_specs=[
            pl.BlockSpec((RS * 16, 128), lambda i: (jnp.minimum(i, nh - 1), 0)),
            pl.BlockSpec((RS * 16, 128), lambda i: (jnp.maximum(i - nh, 0), 0)),
            pl.BlockSpec((RS * 16, 128), lambda i: (i, 0)),
            pl.BlockSpec((RS * 16, 128), lambda i: (i, 0)),
            pl.BlockSpec((RS, 1), lambda i: (i, 0)),
            pl.BlockSpec((RS, 1), lambda i: (i, 0)),
        ],
        out_specs=pl.BlockSpec((RS, hdim), lambda i: (i, 0)),
        out_shape=jax.ShapeDtypeStruct((t, hdim), jnp.float32),
    )(shared_a, shared_b, y0, y1, topk_w[:, 0:1], topk_w[:, 1:2])

    return out.reshape(b, s, hdim).astype(hidden_states.dtype)


# speedup vs baseline: 1.1074x; 1.1074x over previous
import functools

import jax
import jax.numpy as jnp
from jax import lax
from jax.experimental import pallas as pl
from jax.experimental.pallas import tpu as pltpu
from jax.experimental.pallas import tpu_sc as plsc

E = 8
TOP_K = 2
N_GROUP = 4
TOPK_GROUP = 2
SCALE = 2.5

R = 256
RS = 512
NC = 2
NS = 16
NW = NC * NS


def _route(x_flat, gate_weight, bias):
    router_logits = x_flat.astype(jnp.float32) @ gate_weight.T
    scores = jax.nn.sigmoid(router_logits)
    scores_for_choice = scores + bias[None, :]
    t = scores_for_choice.shape[0]
    grp = scores_for_choice.reshape(t, N_GROUP, E // N_GROUP)
    top2_vals, _ = jax.lax.top_k(grp, 2)
    group_scores = top2_vals.sum(axis=-1)
    _, group_idx = jax.lax.top_k(group_scores, TOPK_GROUP)
    ar = jnp.arange(N_GROUP, dtype=group_idx.dtype)[None, :]
    group_mask = jnp.where(
        (group_idx[:, 0:1] == ar) | (group_idx[:, 1:2] == ar), 1.0, 0.0
    ).astype(group_scores.dtype)
    score_mask = jnp.repeat(group_mask, E // N_GROUP, axis=1)
    masked_scores = jnp.where(score_mask > 0, scores_for_choice, 0.0)
    _, topk_idx = jax.lax.top_k(masked_scores, TOP_K)
    topk_w = jnp.take_along_axis(scores, topk_idx, axis=1)
    topk_w = topk_w / (topk_w.sum(axis=-1, keepdims=True) + 1e-20)
    topk_w = topk_w * SCALE
    return topk_idx, topk_w


def _dispatch_sc(x2, i0, i1, p_max):
    t = x2.shape[0] // 16
    per_w = t // NW
    chtok = 32
    n_ch = per_w // chtok
    nj = chtok * 16 // 128
    mesh = plsc.VectorSubcoreMesh(core_axis_name="c", subcore_axis_name="s")

    @functools.partial(
        pl.kernel, mesh=mesh,
        out_type=jax.ShapeDtypeStruct((p_max * 16, 128), jnp.float32),
        scratch_types=[
            pltpu.VMEM((per_w * 16 // 128, 128), jnp.int32),
            pltpu.VMEM((per_w * 16 // 128, 128), jnp.int32),
            pltpu.VMEM((chtok * 16, 128), jnp.float32),
            pltpu.SemaphoreType.DMA,
            pltpu.SemaphoreType.DMA,
        ],
    )
    def run(x_hbm, i0_hbm, i1_hbm, xg_hbm, idx0, idx1, rows, sem0, sem1):
        wid = lax.axis_index("s") * NC + lax.axis_index("c")
        irows = per_w * 16 // 128
        pltpu.sync_copy(i0_hbm.at[pl.ds(wid * irows, irows)], idx0)
        pltpu.sync_copy(i1_hbm.at[pl.ds(wid * irows, irows)], idx1)
        for c in range(n_ch):
            base = (wid * per_w + c * chtok) * 16
            pltpu.sync_copy(x_hbm.at[pl.ds(base, chtok * 16)], rows)
            cps = []
            for j in range(nj):
                seg = rows.at[pl.ds(j * 128, 128)]
                jj = c * nj + j
                cps.append(pltpu.async_copy(seg, xg_hbm.at[idx0.at[jj]], sem0))
                cps.append(pltpu.async_copy(seg, xg_hbm.at[idx1.at[jj]], sem1))
            for cp in cps:
                cp.wait()

    return run(x2, i0, i1)


def _combine_sc(yg2, i0, i1):
    t = i0.shape[0] * 128 // 16
    per_w = t // NW
    chtok = 16
    n_ch = per_w // chtok
    nj = chtok * 16 // 128
    mesh = plsc.VectorSubcoreMesh(core_axis_name="c", subcore_axis_name="s")
    oshape = jax.ShapeDtypeStruct((t * 16, 128), jnp.float32)

    @functools.partial(
        pl.kernel, mesh=mesh,
        out_type=(oshape, oshape),
        scratch_types=[
            pltpu.VMEM((per_w * 16 // 128, 128), jnp.int32),
            pltpu.VMEM((per_w * 16 // 128, 128), jnp.int32),
            pltpu.VMEM((chtok * 16, 128), jnp.float32),
            pltpu.VMEM((chtok * 16, 128), jnp.float32),
            pltpu.SemaphoreType.DMA,
            pltpu.SemaphoreType.DMA,
        ],
    )
    def run(yg_hbm, i0_hbm, i1_hbm, y0_hbm, y1_hbm, idx0, idx1, rows0, rows1,
            sem0, sem1):
        wid = lax.axis_index("s") * NC + lax.axis_index("c")
        irows = per_w * 16 // 128
        pltpu.sync_copy(i0_hbm.at[pl.ds(wid * irows, irows)], idx0)
        pltpu.sync_copy(i1_hbm.at[pl.ds(wid * irows, irows)], idx1)
        for c in range(n_ch):
            base = (wid * per_w + c * chtok) * 16
            cps = []
            for j in range(nj):
                sl = pl.ds(j * 128, 128)
                jj = c * nj + j
                cps.append(
                    pltpu.async_copy(yg_hbm.at[idx0.at[jj]], rows0.at[sl], sem0))
                cps.append(
                    pltpu.async_copy(yg_hbm.at[idx1.at[jj]], rows1.at[sl], sem1))
            for cp in cps:
                cp.wait()
            pltpu.sync_copy(rows0, y0_hbm.at[pl.ds(base, chtok * 16)])
            pltpu.sync_copy(rows1, y1_hbm.at[pl.ds(base, chtok * 16)])

    return run(yg2, i0, i1)


def _reformat_body(x_ref, out_ref):
    out_ref[...] = x_ref[...].reshape(out_ref.shape)


def _gmm_body(te_ref, xg_ref, w1_ref, w3_ref, w2_ref, out_ref):
    rdim = out_ref.shape[0] // 16
    hdim = w1_ref.shape[2]
    xr = xg_ref[...].reshape(rdim, hdim).astype(jnp.bfloat16)
    w1 = w1_ref[0].astype(jnp.bfloat16)
    w3 = w3_ref[0].astype(jnp.bfloat16)
    w2 = w2_ref[0].astype(jnp.bfloat16)
    nt = (((1,), (1,)), ((), ()))
    a = jax.lax.dot_general(xr, w1, nt, preferred_element_type=jnp.float32)
    b = jax.lax.dot_general(xr, w3, nt, preferred_element_type=jnp.float32)
    h = ((a * jax.nn.sigmoid(a)) * b).astype(jnp.bfloat16)
    y = jax.lax.dot_general(h, w2, nt, preferred_element_type=jnp.float32)
    out_ref[...] = y.reshape(out_ref.shape)


def _shared_body(x_ref, ws1_ref, ws3_ref, ws2_ref, out_ref):
    rs, hdim = out_ref.shape
    xr = x_ref[...].reshape(rs, hdim).astype(jnp.bfloat16)
    w1 = ws1_ref[...].astype(jnp.bfloat16)
    w3 = ws3_ref[...].astype(jnp.bfloat16)
    w2 = ws2_ref[...].astype(jnp.bfloat16)
    nt = (((1,), (1,)), ((), ()))
    a = jax.lax.dot_general(xr, w1, nt, preferred_element_type=jnp.float32)
    b = jax.lax.dot_general(xr, w3, nt, preferred_element_type=jnp.float32)
    h = ((a * jax.nn.sigmoid(a)) * b).astype(jnp.bfloat16)
    y = jax.lax.dot_general(h, w2, nt, preferred_element_type=jnp.float32)
    out_ref[...] = y.astype(jnp.bfloat16)


def _add_body(sh_ref, y0_ref, y1_ref, w0_ref, w1c_ref, out_ref):
    rs, hdim = out_ref.shape
    y = sh_ref[...].astype(jnp.float32)
    y = y + w0_ref[...] * y0_ref[...].reshape(rs, hdim)
    y = y + w1c_ref[...] * y1_ref[...].reshape(rs, hdim)
    out_ref[...] = y


def kernel(hidden_states, gate_weight, e_score_correction_bias,
           w1, w2, w3, ws1, ws2, ws3):
    b, s, hdim = hidden_states.shape
    t = b * s
    d_ff = w1.shape[1]
    d_sh = ws1.shape[0]
    x = hidden_states.reshape(t, hdim)

    topk_idx, topk_w = _route(x, gate_weight, e_score_correction_bias)

    p_max = t * TOP_K + E * R
    n_tiles = p_max // R
    onehot = jax.nn.one_hot(topk_idx, E, dtype=jnp.int32).sum(axis=1)
    rank = jnp.cumsum(onehot, axis=0) - onehot
    counts = jnp.sum(onehot, axis=0)
    padded = ((counts + R - 1) // R) * R
    pad_off = jnp.concatenate(
        [jnp.zeros((1,), jnp.int32), jnp.cumsum(padded)[:-1].astype(jnp.int32)])
    slots = jnp.take_along_axis(pad_off[None, :] + rank, topk_idx, axis=1)

    def strip_idx(sv):
        return (sv[:, None] * 16
                + jnp.arange(16, dtype=jnp.int32)[None, :]
                ).astype(jnp.int32).reshape(t * 16 // 128, 128)

    i0 = strip_idx(slots[:, 0])
    i1 = strip_idx(slots[:, 1])
    tile_expert = (jnp.sum(
        pad_off[None, :] <= (jnp.arange(n_tiles, dtype=jnp.int32) * R)[:, None],
        axis=1) - 1).astype(jnp.int32)

    x2 = pl.pallas_call(
        _reformat_body,
        grid=(t // 512,),
        in_specs=[pl.BlockSpec((512, hdim), lambda i: (i, 0))],
        out_specs=pl.BlockSpec((512 * 16, 128), lambda i: (i, 0)),
        out_shape=jax.ShapeDtypeStruct((t * 16, 128), jnp.float32),
    )(x)

    shared2 = pl.pallas_call(
        _shared_body,
        grid=(t // RS,),
        in_specs=[
            pl.BlockSpec((RS * 16, 128), lambda i: (i, 0)),
            pl.BlockSpec((d_sh, hdim), lambda i: (0, 0)),
            pl.BlockSpec((d_sh, hdim), lambda i: (0, 0)),
            pl.BlockSpec((hdim, d_sh), lambda i: (0, 0)),
        ],
        out_specs=pl.BlockSpec((RS, hdim), lambda i: (i, 0)),
        out_shape=jax.ShapeDtypeStruct((t, hdim), jnp.bfloat16),
    )(x2, ws1, ws3, ws2)

    xg2 = _dispatch_sc(x2, i0, i1, p_max)

    yg = pl.pallas_call(
        _gmm_body,
        grid_spec=pltpu.PrefetchScalarGridSpec(
            num_scalar_prefetch=1,
            grid=(n_tiles,),
            in_specs=[
                pl.BlockSpec((R * 16, 128), lambda i, te: (i, 0)),
                pl.BlockSpec((1, d_ff, hdim), lambda i, te: (te[i], 0, 0)),
                pl.BlockSpec((1, d_ff, hdim), lambda i, te: (te[i], 0, 0)),
                pl.BlockSpec((1, hdim, d_ff), lambda i, te: (te[i], 0, 0)),
            ],
            out_specs=pl.BlockSpec((R * 16, 128), lambda i, te: (i, 0)),
        ),
        out_shape=jax.ShapeDtypeStruct((p_max * 16, 128), jnp.float32),
    )(tile_expert, xg2, w1, w3, w2)

    y0, y1 = _combine_sc(yg, i0, i1)

    out = pl.pallas_call(
        _add_body,
        grid=(t // RS,),
        in_specs=[
            pl.BlockSpec((RS, hdim), lambda i: (i, 0)),
            pl.BlockSpec((RS * 16, 128), lambda i: (i, 0)),
            pl.BlockSpec((RS * 16, 128), lambda i: (i, 0)),
            pl.BlockSpec((RS, 1), lambda i: (i, 0)),
            pl.BlockSpec((RS, 1), lambda i: (i, 0)),
        ],
        out_specs=pl.BlockSpec((RS, hdim), lambda i: (i, 0)),
        out_shape=jax.ShapeDtypeStruct((t, hdim), jnp.float32),
    )(shared2, y0, y1, topk_w[:, 0:1], topk_w[:, 1:2])

    return out.reshape(b, s, hdim).astype(hidden_states.dtype)
